# baseline (speedup 1.0000x reference)
import functools
import math

import jax
import jax.numpy as jnp
from jax import lax
from jax.experimental import pallas as pl
from jax.experimental.pallas import tpu as pltpu
from jax.experimental.pallas import tpu_sc as plsc

D = 64
LANES = 16
CHUNK = 128
NBUF = 4
NC, NS = 2, 16
NW = NC * NS


def _emb_sc(x_rows, table):
    n_rows = x_rows.shape[0]
    rows_per_w = n_rows // NW
    ngroups = rows_per_w // NBUF
    scale = jnp.float32(math.sqrt(float(D)))
    mesh = plsc.VectorSubcoreMesh(core_axis_name="c", subcore_axis_name="s")

    @functools.partial(
        pl.kernel,
        out_type=jax.ShapeDtypeStruct((n_rows * CHUNK, D), jnp.float32),
        mesh=mesh,
        scratch_types=[
            pltpu.VMEM((rows_per_w, CHUNK), jnp.int32),
            pltpu.VMEM((NBUF, CHUNK, D), jnp.float32),
            pltpu.VMEM((NBUF, CHUNK, D), jnp.float32),
            pltpu.SemaphoreType.DMA,
            pltpu.SemaphoreType.DMA,
        ],
        compiler_params=pltpu.CompilerParams(use_tc_tiling_on_sc=False),
    )
    def k(x_hbm, table_hbm, out_hbm, idx_v, inb, outb, gsem, osem):
        c = lax.axis_index("c")
        s = lax.axis_index("s")
        wid = s * NC + c
        row0 = wid * rows_per_w
        pltpu.sync_copy(x_hbm.at[pl.ds(row0, rows_per_w)], idx_v)

        def gather(j, b):
            pltpu.async_copy(table_hbm.at[idx_v.at[j]], inb.at[b], gsem)

        def gather_wait(j, b):
            del j
            pltpu.make_async_copy(
                table_hbm.at[pl.ds(0, CHUNK)], inb.at[b], gsem
            ).wait()

        def out_copy(j, b):
            pltpu.async_copy(
                outb.at[b], out_hbm.at[pl.ds((row0 + j) * CHUNK, CHUNK)], osem
            )

        def out_wait(j, b):
            pltpu.make_async_copy(
                outb.at[b], out_hbm.at[pl.ds((row0 + j) * CHUNK, CHUNK)], osem
            ).wait()

        for b in range(NBUF):
            gather(b, b)

        def group(g, carry):
            j0 = g * NBUF
            for b in range(NBUF):
                j = j0 + b
                gather_wait(j, b)

                @pl.when(g > 0)
                def _(b=b, j=j):
                    out_wait(j - NBUF, b)

                def mul(rq, carry, b=b):
                    for u in range(4):
                        r = rq * 4 + u
                        for kk in range(D // LANES):
                            sl = pl.ds(kk * LANES, LANES)
                            outb[b, r, sl] = inb[b, r, sl] * scale
                    return carry

                lax.fori_loop(0, CHUNK // 4, mul, 0)

                out_copy(j, b)

                @pl.when(g + 1 < ngroups)
                def _(b=b, j=j):
                    gather(j + NBUF, b)

            return carry

        lax.fori_loop(0, ngroups, group, 0)

        for b in range(NBUF):
            out_wait((ngroups - 1) * NBUF + b, b)

    return k(x_rows, table)


def kernel(x, table):
    b, s = x.shape
    n = b * s
    x_rows = x.reshape(n // CHUNK, CHUNK).astype(jnp.int32)
    out = _emb_sc(x_rows, table)
    return out.reshape(b, s, D)
